# Initial kernel scaffold; baseline (speedup 1.0000x reference)
import jax
import jax.numpy as jnp
from jax.experimental import pallas as pl


def kernel(vectors, x, V, senders, W1, W2a, b2a, W2b, b2b, W2c, b2c, Wlin):
    raise NotImplementedError("write your pallas kernel here")


# speedup vs baseline: 16.8734x; 16.8734x over previous
import functools

import numpy as np
import jax
import jax.numpy as jnp
from jax import lax
from jax.experimental import pallas as pl
from jax.experimental.pallas import tpu as pltpu
from jax.experimental.pallas import tpu_sc as plsc

N_NODES = 10000
E = 160000
D_FEAT = 128
MUL = 32
H = 256
EPSILON = 0.25
CUTOFF = 2.0
F4 = 4 * MUL

_Rm = np.zeros((MUL, F4), np.float32)
_U3s = np.zeros((3, F4), np.float32)
_mask1 = np.zeros((1, F4), np.float32)
_Smat = np.zeros((F4, 2 * MUL), np.float32)
_B4 = np.zeros((F4, F4), np.float32)
for _i in range(MUL):
    _mask1[0, 4 * _i] = 1.0
    _Smat[4 * _i, _i] = 1.0
    for _k in range(4):
        _Rm[_i, 4 * _i + _k] = 1.0
        _B4[4 * _i, 4 * _i + _k] = 1.0
    for _c in range(3):
        _U3s[_c, 4 * _i + 1 + _c] = np.sqrt(3.0)
        _Smat[4 * _i + 1 + _c, MUL + _i] = 1.0 / np.sqrt(3.0)

_G_ROWS = np.array([4 * i + 1 + c for c in range(3) for i in range(MUL)])
_G_COLS = np.array([3 * o + c for c in range(3) for o in range(MUL)])

_P = 6
_C1 = (_P + 1.0) * (_P + 2.0) / 2.0
_C2 = _P * (_P + 2.0)
_C3 = _P * (_P + 1.0) / 2.0

_B1 = 1600


def _tc1_body(vec_ref, x_ref, w1r_ref, u3s_ref, m1_ref, msg_ref, env_ref):
    v = vec_ref[...]
    l2 = jnp.sum(v * v, axis=1, keepdims=True)
    ln = jnp.sqrt(l2)
    rinv = 1.0 / (ln + 1e-9)
    yt = jnp.dot(v, u3s_ref[...], preferred_element_type=jnp.float32) * rinv
    yt = yt + m1_ref[...]
    w4 = jnp.dot(x_ref[...], w1r_ref[...], preferred_element_type=jnp.float32)
    msg_ref[...] = w4 * yt
    dn = ln * (1.0 / CUTOFF)
    d2 = dn * dn
    d6 = d2 * d2 * d2
    env = jnp.where(dn < 1.0, 1.0 - _C1 * d6 + _C2 * d6 * dn - _C3 * d6 * d2, 0.0)
    env_ref[...] = env


def _tc1(vectors, x, w1r):
    grid = (E // _B1,)
    return pl.pallas_call(
        _tc1_body,
        grid=grid,
        in_specs=[
            pl.BlockSpec((_B1, 3), lambda i: (i, 0)),
            pl.BlockSpec((_B1, D_FEAT), lambda i: (i, 0)),
            pl.BlockSpec((D_FEAT, F4), lambda i: (0, 0)),
            pl.BlockSpec((3, F4), lambda i: (0, 0)),
            pl.BlockSpec((1, F4), lambda i: (0, 0)),
        ],
        out_specs=[
            pl.BlockSpec((_B1, F4), lambda i: (i, 0)),
            pl.BlockSpec((_B1, 1), lambda i: (i, 0)),
        ],
        out_shape=[
            jax.ShapeDtypeStruct((E, F4), jnp.float32),
            jax.ShapeDtypeStruct((E, 1), jnp.float32),
        ],
    )(vectors, x, w1r, jnp.asarray(_U3s), jnp.asarray(_mask1))


_CHUNK = 128
_NCHUNK = E // _CHUNK
_NC = 2
_NS = 16
_NW = _NC * _NS
_TRIPS = -(-_NCHUNK // _NW)
_NPAD = 10240
_STRIPE = _NPAD // _NS


def _sc_scatter_body(msg_hbm, send_hbm, zeros_hbm, table_hbm,
                     idx_v, rows_v, table_sh, sem):
    c = lax.axis_index("c")
    s = lax.axis_index("s")
    wid = s * _NC + c
    pltpu.sync_copy(zeros_hbm, rows_v)

    def zbody(q, carry):
        pltpu.sync_copy(rows_v, table_sh.at[pl.ds(s * _STRIPE + q * _CHUNK, _CHUNK)])
        return carry

    lax.fori_loop(0, _STRIPE // _CHUNK, zbody, 0)
    plsc.subcore_barrier()

    def body(t, carry):
        j = wid + t * _NW

        @pl.when(j < _NCHUNK)
        def _():
            base = j * _CHUNK
            pltpu.sync_copy(send_hbm.at[pl.ds(base, _CHUNK)], idx_v)
            pltpu.sync_copy(msg_hbm.at[pl.ds(base, _CHUNK)], rows_v)
            pltpu.sync_copy(rows_v, table_sh.at[idx_v], add=True)

        return carry

    lax.fori_loop(0, _TRIPS, body, 0)
    plsc.subcore_barrier()

    def dbody(q, carry):
        pltpu.sync_copy(table_sh.at[pl.ds(s * _STRIPE + q * _CHUNK, _CHUNK)], rows_v)
        pltpu.sync_copy(rows_v,
                        table_hbm.at[pl.ds(c * _NPAD + s * _STRIPE + q * _CHUNK,
                                           _CHUNK)])
        return carry

    lax.fori_loop(0, _STRIPE // _CHUNK, dbody, 0)


def _sc_scatter(msg, senders, zeros_stripe):
    mesh = plsc.VectorSubcoreMesh(core_axis_name="c", subcore_axis_name="s")
    f = functools.partial(
        pl.kernel,
        mesh=mesh,
        out_type=jax.ShapeDtypeStruct((_NC * _NPAD, F4), jnp.float32),
        scratch_types=[
            pltpu.VMEM((_CHUNK,), jnp.int32),
            pltpu.VMEM((_CHUNK, F4), jnp.float32),
            pltpu.VMEM_SHARED((_NPAD, F4), jnp.float32),
            pltpu.SemaphoreType.DMA,
        ],
    )(_sc_scatter_body)
    return f(msg, senders, zeros_stripe)


def _sc_gather_body(table_hbm, send_hbm, wy_hbm, idx_v, rows_v, sem):
    c = lax.axis_index("c")
    s = lax.axis_index("s")
    wid = s * _NC + c

    def body(t, carry):
        j = wid + t * _NW

        @pl.when(j < _NCHUNK)
        def _():
            base = j * _CHUNK
            pltpu.sync_copy(send_hbm.at[pl.ds(base, _CHUNK)], idx_v)
            pltpu.async_copy(table_hbm.at[idx_v], rows_v, sem).wait()
            pltpu.sync_copy(rows_v, wy_hbm.at[pl.ds(base, _CHUNK)])

        return carry

    lax.fori_loop(0, _TRIPS, body, 0)


def _sc_gather(table, senders):
    mesh = plsc.VectorSubcoreMesh(core_axis_name="c", subcore_axis_name="s")
    f = functools.partial(
        pl.kernel,
        mesh=mesh,
        out_type=jax.ShapeDtypeStruct((E, F4), jnp.float32),
        scratch_types=[
            pltpu.VMEM((_CHUNK,), jnp.int32),
            pltpu.VMEM((_CHUNK, F4), jnp.float32),
            pltpu.SemaphoreType.DMA,
        ],
    )(_sc_gather_body)
    return f(table, senders)


_BN = 2048


def _tc2_body(a_ref, b_ref, o_ref):
    o_ref[...] = a_ref[...] + b_ref[...]


def _tc2(tables):
    nb = _NPAD // _BN
    return pl.pallas_call(
        _tc2_body,
        grid=(nb,),
        in_specs=[
            pl.BlockSpec((_BN, F4), lambda i: (i, 0)),
            pl.BlockSpec((_BN, F4), lambda i, nb=nb: (i + nb, 0)),
        ],
        out_specs=pl.BlockSpec((_BN, F4), lambda i: (i, 0)),
        out_shape=jax.ShapeDtypeStruct((_NPAD, F4), jnp.float32),
    )(tables, tables)


_B3 = 1600


def _tc3_body(x_ref, v_ref, wy_ref, env_ref, w2ax_ref, w2ap_ref, b2a_ref,
              w2b_ref, b2b_ref, w2c_ref, b2c_ref, b4_ref, g1_ref, g2_ref,
              xo_ref, vo_ref):
    xv = x_ref[...]
    Vv = v_ref[...]
    wy = wy_ref[...]
    P = wy * Vv
    h = (jnp.dot(xv, w2ax_ref[...], preferred_element_type=jnp.float32)
         + jnp.dot(P, w2ap_ref[...], preferred_element_type=jnp.float32)
         + b2a_ref[...])
    h = h * jax.nn.sigmoid(h)
    h = jnp.dot(h, w2b_ref[...], preferred_element_type=jnp.float32) + b2b_ref[...]
    h = h * jax.nn.sigmoid(h)
    h = jnp.dot(h, w2c_ref[...], preferred_element_type=jnp.float32) + b2c_ref[...]
    xo_ref[...] = env_ref[...] * h
    t1 = jnp.dot(wy, b4_ref[...], preferred_element_type=jnp.float32) * Vv
    t2 = wy * jnp.dot(Vv, b4_ref[...], preferred_element_type=jnp.float32)
    vo_ref[...] = (jnp.dot(t1, g1_ref[...], preferred_element_type=jnp.float32)
                   + jnp.dot(t2, g2_ref[...], preferred_element_type=jnp.float32))


def _tc3(x, V, wy, env, w2ax, w2ap, b2a, w2b, b2b, w2c, b2c, g1, g2):
    grid = (E // _B3,)
    zero = lambda i: (0, 0)
    return pl.pallas_call(
        _tc3_body,
        grid=grid,
        in_specs=[
            pl.BlockSpec((_B3, D_FEAT), lambda i: (i, 0)),
            pl.BlockSpec((_B3, F4), lambda i: (i, 0)),
            pl.BlockSpec((_B3, F4), lambda i: (i, 0)),
            pl.BlockSpec((_B3, 1), lambda i: (i, 0)),
            pl.BlockSpec((D_FEAT, H), zero),
            pl.BlockSpec((F4, H), zero),
            pl.BlockSpec((1, H), zero),
            pl.BlockSpec((H, H), zero),
            pl.BlockSpec((1, H), zero),
            pl.BlockSpec((H, H), zero),
            pl.BlockSpec((1, H), zero),
            pl.BlockSpec((F4, F4), zero),
            pl.BlockSpec((F4, 3 * MUL), zero),
            pl.BlockSpec((F4, 3 * MUL), zero),
        ],
        out_specs=[
            pl.BlockSpec((_B3, H), lambda i: (i, 0)),
            pl.BlockSpec((_B3, 3 * MUL), lambda i: (i, 0)),
        ],
        out_shape=[
            jax.ShapeDtypeStruct((E, H), jnp.float32),
            jax.ShapeDtypeStruct((E, 3 * MUL), jnp.float32),
        ],
    )(x, V, wy, env, w2ax, w2ap, b2a, w2b, b2b, w2c, b2c,
      jnp.asarray(_B4), g1, g2)


def kernel(vectors, x, V, senders, W1, W2a, b2a, W2b, b2b, W2c, b2c, Wlin):
    w1r = W1 @ jnp.asarray(_Rm)
    w2ax = W2a[:D_FEAT]
    w2ap = EPSILON * (jnp.asarray(_Smat) @ W2a[D_FEAT:])
    scale = EPSILON / np.sqrt(2.0 * MUL)
    wl1 = (Wlin[:MUL] * scale)
    wl2 = (Wlin[MUL:] * scale)
    g1 = jnp.zeros((F4, 3 * MUL), jnp.float32)
    g2 = jnp.zeros((F4, 3 * MUL), jnp.float32)
    rows = _G_ROWS.reshape(3, MUL)
    cols = _G_COLS.reshape(3, MUL)
    for ci in range(3):
        g1 = g1.at[rows[ci][:, None], cols[ci][None, :]].set(wl1)
        g2 = g2.at[rows[ci][:, None], cols[ci][None, :]].set(wl2)
    b2a2 = b2a.reshape(1, H)
    b2b2 = b2b.reshape(1, H)
    b2c2 = b2c.reshape(1, H)
    zeros_stripe = jnp.zeros((_CHUNK, F4), jnp.float32)

    msg, env = _tc1(vectors, x, w1r)
    tables = _sc_scatter(msg, senders, zeros_stripe)
    agg = _tc2(tables)
    wy = _sc_gather(agg, senders)
    x_out, v_out = _tc3(x, V, wy, env, w2ax, w2ap, b2a2, W2b, b2b2,
                        W2c, b2c2, g1, g2)
    return (x_out, v_out)


# speedup vs baseline: 18.4111x; 1.0911x over previous
import functools

import numpy as np
import jax
import jax.numpy as jnp
from jax import lax
from jax.experimental import pallas as pl
from jax.experimental.pallas import tpu as pltpu
from jax.experimental.pallas import tpu_sc as plsc

N_NODES = 10000
E = 160000
D_FEAT = 128
MUL = 32
H = 256
EPSILON = 0.25
CUTOFF = 2.0
F4 = 4 * MUL

_Rm = np.zeros((MUL, F4), np.float32)
_U3s = np.zeros((3, F4), np.float32)
_mask1 = np.zeros((1, F4), np.float32)
_Smat = np.zeros((F4, 2 * MUL), np.float32)
_B4 = np.zeros((F4, F4), np.float32)
for _i in range(MUL):
    _mask1[0, 4 * _i] = 1.0
    _Smat[4 * _i, _i] = 1.0
    for _k in range(4):
        _Rm[_i, 4 * _i + _k] = 1.0
        _B4[4 * _i, 4 * _i + _k] = 1.0
    for _c in range(3):
        _U3s[_c, 4 * _i + 1 + _c] = np.sqrt(3.0)
        _Smat[4 * _i + 1 + _c, MUL + _i] = 1.0 / np.sqrt(3.0)

_G_ROWS = np.array([4 * i + 1 + c for c in range(3) for i in range(MUL)])
_G_COLS = np.array([3 * o + c for c in range(3) for o in range(MUL)])

_P = 6
_C1 = (_P + 1.0) * (_P + 2.0) / 2.0
_C2 = _P * (_P + 2.0)
_C3 = _P * (_P + 1.0) / 2.0

_B1 = 3200


def _tc1_body(vec_ref, x_ref, w1r_ref, u3s_ref, m1_ref, msg_ref, env_ref):
    v = vec_ref[...]
    l2 = jnp.sum(v * v, axis=1, keepdims=True)
    ln = jnp.sqrt(l2)
    rinv = 1.0 / (ln + 1e-9)
    yt = jnp.dot(v, u3s_ref[...], preferred_element_type=jnp.float32) * rinv
    yt = yt + m1_ref[...]
    w4 = jnp.dot(x_ref[...], w1r_ref[...], preferred_element_type=jnp.float32)
    msg_ref[...] = w4 * yt
    dn = ln * (1.0 / CUTOFF)
    d2 = dn * dn
    d6 = d2 * d2 * d2
    env = jnp.where(dn < 1.0, 1.0 - _C1 * d6 + _C2 * d6 * dn - _C3 * d6 * d2, 0.0)
    env_ref[...] = env


def _tc1(vectors, x, w1r):
    grid = (E // _B1,)
    return pl.pallas_call(
        _tc1_body,
        grid=grid,
        in_specs=[
            pl.BlockSpec((_B1, 3), lambda i: (i, 0)),
            pl.BlockSpec((_B1, D_FEAT), lambda i: (i, 0)),
            pl.BlockSpec((D_FEAT, F4), lambda i: (0, 0)),
            pl.BlockSpec((3, F4), lambda i: (0, 0)),
            pl.BlockSpec((1, F4), lambda i: (0, 0)),
        ],
        out_specs=[
            pl.BlockSpec((_B1, F4), lambda i: (i, 0)),
            pl.BlockSpec((_B1, 1), lambda i: (i, 0)),
        ],
        out_shape=[
            jax.ShapeDtypeStruct((E, F4), jnp.float32),
            jax.ShapeDtypeStruct((E, 1), jnp.float32),
        ],
    )(vectors, x, w1r, jnp.asarray(_U3s), jnp.asarray(_mask1))


_CHUNK = 128
_NCHUNK = E // _CHUNK
_NC = 2
_NS = 16
_NW = _NC * _NS
_TRIPS = -(-_NCHUNK // _NW)
_NPAD = 10240
_STRIPE = _NPAD // _NS


def _sc_scatter_body(msg_hbm, send_hbm, zeros_hbm, table_hbm,
                     idx_v, rows_v, table_sh, sem):
    c = lax.axis_index("c")
    s = lax.axis_index("s")
    wid = s * _NC + c
    pltpu.sync_copy(zeros_hbm, rows_v)

    def zbody(q, carry):
        pltpu.sync_copy(rows_v, table_sh.at[pl.ds(s * _STRIPE + q * _CHUNK, _CHUNK)])
        return carry

    lax.fori_loop(0, _STRIPE // _CHUNK, zbody, 0)
    plsc.subcore_barrier()

    def body(t, carry):
        j = wid + t * _NW

        @pl.when(j < _NCHUNK)
        def _():
            base = j * _CHUNK
            pltpu.sync_copy(send_hbm.at[pl.ds(base, _CHUNK)], idx_v)
            pltpu.sync_copy(msg_hbm.at[pl.ds(base, _CHUNK)], rows_v)
            pltpu.sync_copy(rows_v, table_sh.at[idx_v], add=True)

        return carry

    lax.fori_loop(0, _TRIPS, body, 0)
    plsc.subcore_barrier()

    def dbody(q, carry):
        pltpu.sync_copy(table_sh.at[pl.ds(s * _STRIPE + q * _CHUNK, _CHUNK)], rows_v)
        pltpu.sync_copy(rows_v,
                        table_hbm.at[pl.ds(c * _NPAD + s * _STRIPE + q * _CHUNK,
                                           _CHUNK)])
        return carry

    lax.fori_loop(0, _STRIPE // _CHUNK, dbody, 0)


def _sc_scatter(msg, senders, zeros_stripe):
    mesh = plsc.VectorSubcoreMesh(core_axis_name="c", subcore_axis_name="s")
    f = functools.partial(
        pl.kernel,
        mesh=mesh,
        out_type=jax.ShapeDtypeStruct((_NC * _NPAD, F4), jnp.float32),
        scratch_types=[
            pltpu.VMEM((_CHUNK,), jnp.int32),
            pltpu.VMEM((_CHUNK, F4), jnp.float32),
            pltpu.VMEM_SHARED((_NPAD, F4), jnp.float32),
            pltpu.SemaphoreType.DMA,
        ],
    )(_sc_scatter_body)
    return f(msg, senders, zeros_stripe)


def _sc_gather_body(table_hbm, send_hbm, wy_hbm, idx_v, rows_v, sem):
    c = lax.axis_index("c")
    s = lax.axis_index("s")
    wid = s * _NC + c

    def body(t, carry):
        j = wid + t * _NW

        @pl.when(j < _NCHUNK)
        def _():
            base = j * _CHUNK
            pltpu.sync_copy(send_hbm.at[pl.ds(base, _CHUNK)], idx_v)
            pltpu.async_copy(table_hbm.at[idx_v], rows_v, sem).wait()
            pltpu.sync_copy(rows_v, wy_hbm.at[pl.ds(base, _CHUNK)])

        return carry

    lax.fori_loop(0, _TRIPS, body, 0)


def _sc_gather(table, senders):
    mesh = plsc.VectorSubcoreMesh(core_axis_name="c", subcore_axis_name="s")
    f = functools.partial(
        pl.kernel,
        mesh=mesh,
        out_type=jax.ShapeDtypeStruct((E, F4), jnp.float32),
        scratch_types=[
            pltpu.VMEM((_CHUNK,), jnp.int32),
            pltpu.VMEM((_CHUNK, F4), jnp.float32),
            pltpu.SemaphoreType.DMA,
        ],
    )(_sc_gather_body)
    return f(table, senders)


_BN = 2048


def _tc2_body(a_ref, b_ref, o_ref):
    o_ref[...] = a_ref[...] + b_ref[...]


def _tc2(tables):
    nb = _NPAD // _BN
    return pl.pallas_call(
        _tc2_body,
        grid=(nb,),
        in_specs=[
            pl.BlockSpec((_BN, F4), lambda i: (i, 0)),
            pl.BlockSpec((_BN, F4), lambda i, nb=nb: (i + nb, 0)),
        ],
        out_specs=pl.BlockSpec((_BN, F4), lambda i: (i, 0)),
        out_shape=jax.ShapeDtypeStruct((_NPAD, F4), jnp.float32),
    )(tables, tables)


_B3 = 3200


def _tc3_body(x_ref, v_ref, wy_ref, env_ref, w2ax_ref, w2ap_ref, b2a_ref,
              w2b_ref, b2b_ref, w2c_ref, b2c_ref, b4_ref, g1_ref, g2_ref,
              xo_ref, vo_ref):
    xv = x_ref[...]
    Vv = v_ref[...]
    wy = wy_ref[...]
    P = wy * Vv
    h = (jnp.dot(xv, w2ax_ref[...], preferred_element_type=jnp.float32)
         + jnp.dot(P, w2ap_ref[...], preferred_element_type=jnp.float32)
         + b2a_ref[...])
    h = h * jax.nn.sigmoid(h)
    h = jnp.dot(h, w2b_ref[...], preferred_element_type=jnp.float32) + b2b_ref[...]
    h = h * jax.nn.sigmoid(h)
    h = jnp.dot(h, w2c_ref[...], preferred_element_type=jnp.float32) + b2c_ref[...]
    xo_ref[...] = env_ref[...] * h
    t1 = jnp.dot(wy, b4_ref[...], preferred_element_type=jnp.float32) * Vv
    t2 = wy * jnp.dot(Vv, b4_ref[...], preferred_element_type=jnp.float32)
    vo_ref[...] = (jnp.dot(t1, g1_ref[...], preferred_element_type=jnp.float32)
                   + jnp.dot(t2, g2_ref[...], preferred_element_type=jnp.float32))


def _tc3(x, V, wy, env, w2ax, w2ap, b2a, w2b, b2b, w2c, b2c, g1, g2):
    grid = (E // _B3,)
    zero = lambda i: (0, 0)
    return pl.pallas_call(
        _tc3_body,
        grid=grid,
        in_specs=[
            pl.BlockSpec((_B3, D_FEAT), lambda i: (i, 0)),
            pl.BlockSpec((_B3, F4), lambda i: (i, 0)),
            pl.BlockSpec((_B3, F4), lambda i: (i, 0)),
            pl.BlockSpec((_B3, 1), lambda i: (i, 0)),
            pl.BlockSpec((D_FEAT, H), zero),
            pl.BlockSpec((F4, H), zero),
            pl.BlockSpec((1, H), zero),
            pl.BlockSpec((H, H), zero),
            pl.BlockSpec((1, H), zero),
            pl.BlockSpec((H, H), zero),
            pl.BlockSpec((1, H), zero),
            pl.BlockSpec((F4, F4), zero),
            pl.BlockSpec((F4, 3 * MUL), zero),
            pl.BlockSpec((F4, 3 * MUL), zero),
        ],
        out_specs=[
            pl.BlockSpec((_B3, H), lambda i: (i, 0)),
            pl.BlockSpec((_B3, 3 * MUL), lambda i: (i, 0)),
        ],
        out_shape=[
            jax.ShapeDtypeStruct((E, H), jnp.float32),
            jax.ShapeDtypeStruct((E, 3 * MUL), jnp.float32),
        ],
    )(x, V, wy, env, w2ax, w2ap, b2a, w2b, b2b, w2c, b2c,
      jnp.asarray(_B4), g1, g2)


def kernel(vectors, x, V, senders, W1, W2a, b2a, W2b, b2b, W2c, b2c, Wlin):
    w1r = W1 @ jnp.asarray(_Rm)
    w2ax = W2a[:D_FEAT]
    w2ap = EPSILON * (jnp.asarray(_Smat) @ W2a[D_FEAT:])
    scale = EPSILON / np.sqrt(2.0 * MUL)
    wl1 = (Wlin[:MUL] * scale)
    wl2 = (Wlin[MUL:] * scale)
    g1 = jnp.zeros((F4, 3 * MUL), jnp.float32)
    g2 = jnp.zeros((F4, 3 * MUL), jnp.float32)
    rows = _G_ROWS.reshape(3, MUL)
    cols = _G_COLS.reshape(3, MUL)
    for ci in range(3):
        g1 = g1.at[rows[ci][:, None], cols[ci][None, :]].set(wl1)
        g2 = g2.at[rows[ci][:, None], cols[ci][None, :]].set(wl2)
    b2a2 = b2a.reshape(1, H)
    b2b2 = b2b.reshape(1, H)
    b2c2 = b2c.reshape(1, H)
    zeros_stripe = jnp.zeros((_CHUNK, F4), jnp.float32)

    msg, env = _tc1(vectors, x, w1r)
    tables = _sc_scatter(msg, senders, zeros_stripe)
    agg = _tc2(tables)
    wy = _sc_gather(agg, senders)
    x_out, v_out = _tc3(x, V, wy, env, w2ax, w2ap, b2a2, W2b, b2b2,
                        W2c, b2c2, g1, g2)
    return (x_out, v_out)


# speedup vs baseline: 20.5151x; 1.1143x over previous
import functools

import numpy as np
import jax
import jax.numpy as jnp
from jax import lax
from jax.experimental import pallas as pl
from jax.experimental.pallas import tpu as pltpu
from jax.experimental.pallas import tpu_sc as plsc

N_NODES = 10000
E = 160000
D_FEAT = 128
MUL = 32
H = 256
EPSILON = 0.25
CUTOFF = 2.0
F4 = 4 * MUL

_Rm = np.zeros((MUL, F4), np.float32)
_U3s = np.zeros((3, F4), np.float32)
_mask1 = np.zeros((1, F4), np.float32)
_Smat = np.zeros((F4, 2 * MUL), np.float32)
_B4 = np.zeros((F4, F4), np.float32)
for _i in range(MUL):
    _mask1[0, 4 * _i] = 1.0
    _Smat[4 * _i, _i] = 1.0
    for _k in range(4):
        _Rm[_i, 4 * _i + _k] = 1.0
        _B4[4 * _i, 4 * _i + _k] = 1.0
    for _c in range(3):
        _U3s[_c, 4 * _i + 1 + _c] = np.sqrt(3.0)
        _Smat[4 * _i + 1 + _c, MUL + _i] = 1.0 / np.sqrt(3.0)

_G_ROWS = np.array([4 * i + 1 + c for c in range(3) for i in range(MUL)])
_G_COLS = np.array([3 * o + c for c in range(3) for o in range(MUL)])

_P = 6
_C1 = (_P + 1.0) * (_P + 2.0) / 2.0
_C2 = _P * (_P + 2.0)
_C3 = _P * (_P + 1.0) / 2.0

_B1 = 3200


def _tc1_body(vec_ref, x_ref, w1r_ref, u3s_ref, m1_ref, msg_ref, env_ref):
    v = vec_ref[...]
    l2 = jnp.sum(v * v, axis=1, keepdims=True)
    ln = jnp.sqrt(l2)
    rinv = 1.0 / (ln + 1e-9)
    yt = jnp.dot(v, u3s_ref[...], preferred_element_type=jnp.float32) * rinv
    yt = yt + m1_ref[...]
    w4 = jnp.dot(x_ref[...], w1r_ref[...], preferred_element_type=jnp.float32)
    msg_ref[...] = w4 * yt
    dn = ln * (1.0 / CUTOFF)
    d2 = dn * dn
    d6 = d2 * d2 * d2
    env = jnp.where(dn < 1.0, 1.0 - _C1 * d6 + _C2 * d6 * dn - _C3 * d6 * d2, 0.0)
    env_ref[...] = env


def _tc1(vectors, x, w1r):
    grid = (E // _B1,)
    return pl.pallas_call(
        _tc1_body,
        grid=grid,
        in_specs=[
            pl.BlockSpec((_B1, 3), lambda i: (i, 0)),
            pl.BlockSpec((_B1, D_FEAT), lambda i: (i, 0)),
            pl.BlockSpec((D_FEAT, F4), lambda i: (0, 0)),
            pl.BlockSpec((3, F4), lambda i: (0, 0)),
            pl.BlockSpec((1, F4), lambda i: (0, 0)),
        ],
        out_specs=[
            pl.BlockSpec((_B1, F4), lambda i: (i, 0)),
            pl.BlockSpec((_B1, 1), lambda i: (i, 0)),
        ],
        out_shape=[
            jax.ShapeDtypeStruct((E, F4), jnp.float32),
            jax.ShapeDtypeStruct((E, 1), jnp.float32),
        ],
    )(vectors, x, w1r, jnp.asarray(_U3s), jnp.asarray(_mask1))


_CHUNK = 128
_NCHUNK = E // _CHUNK
_NC = 2
_NS = 16
_NW = _NC * _NS
_TRIPS = -(-_NCHUNK // _NW)
_NPAD = 10240
_STRIPE = _NPAD // _NS


def _load_all_idx(send_hbm, idx2d, sem_idx, wid):
    def ibody(t, carry):
        jc = jnp.minimum(wid + t * _NW, _NCHUNK - 1)
        pltpu.async_copy(send_hbm.at[pl.ds(jc * _CHUNK, _CHUNK)],
                         idx2d.at[t], sem_idx)
        return carry

    lax.fori_loop(0, _TRIPS, ibody, 0)

    def iwait(t, carry):
        pltpu.make_async_copy(send_hbm.at[pl.ds(0, _CHUNK)],
                              idx2d.at[t], sem_idx).wait()
        return carry

    lax.fori_loop(0, _TRIPS, iwait, 0)


def _sc_scatter_body(msg_hbm, send_hbm, zeros_hbm, table_hbm,
                     idx2d, rows_v, table_sh, sem_idx, sem_in):
    c = lax.axis_index("c")
    s = lax.axis_index("s")
    wid = s * _NC + c

    pltpu.sync_copy(zeros_hbm, rows_v.at[0])

    def zbody(q, carry):
        pltpu.sync_copy(rows_v.at[0],
                        table_sh.at[pl.ds(s * _STRIPE + q * _CHUNK, _CHUNK)])
        return carry

    lax.fori_loop(0, _STRIPE // _CHUNK, zbody, 0)
    _load_all_idx(send_hbm, idx2d, sem_idx, wid)
    plsc.subcore_barrier()

    def start_in(t):
        jc = jnp.minimum(wid + t * _NW, _NCHUNK - 1)
        pltpu.async_copy(msg_hbm.at[pl.ds(jc * _CHUNK, _CHUNK)],
                         rows_v.at[t % 2], sem_in)

    start_in(0)

    def body(t, carry):
        b = t % 2
        pltpu.make_async_copy(msg_hbm.at[pl.ds(0, _CHUNK)],
                              rows_v.at[b], sem_in).wait()

        @pl.when(t + 1 < _TRIPS)
        def _():
            start_in(t + 1)

        @pl.when(wid + t * _NW < _NCHUNK)
        def _():
            pltpu.sync_copy(rows_v.at[b], table_sh.at[idx2d.at[t]], add=True)

        return carry

    lax.fori_loop(0, _TRIPS, body, 0)
    plsc.subcore_barrier()

    def dbody(q, carry):
        pltpu.sync_copy(table_sh.at[pl.ds(s * _STRIPE + q * _CHUNK, _CHUNK)],
                        rows_v.at[0])
        pltpu.sync_copy(rows_v.at[0],
                        table_hbm.at[pl.ds(c * _NPAD + s * _STRIPE + q * _CHUNK,
                                           _CHUNK)])
        return carry

    lax.fori_loop(0, _STRIPE // _CHUNK, dbody, 0)


def _sc_scatter(msg, senders, zeros_stripe):
    mesh = plsc.VectorSubcoreMesh(core_axis_name="c", subcore_axis_name="s")
    f = functools.partial(
        pl.kernel,
        mesh=mesh,
        out_type=jax.ShapeDtypeStruct((_NC * _NPAD, F4), jnp.float32),
        scratch_types=[
            pltpu.VMEM((_TRIPS, _CHUNK), jnp.int32),
            pltpu.VMEM((2, _CHUNK, F4), jnp.float32),
            pltpu.VMEM_SHARED((_NPAD, F4), jnp.float32),
            pltpu.SemaphoreType.DMA,
            pltpu.SemaphoreType.DMA,
        ],
    )(_sc_scatter_body)
    return f(msg, senders, zeros_stripe)


def _sc_gather_body(table_hbm, send_hbm, wy_hbm, idx2d, rows_v,
                    sem_idx, sem_g, sem_s):
    c = lax.axis_index("c")
    s = lax.axis_index("s")
    wid = s * _NC + c
    _load_all_idx(send_hbm, idx2d, sem_idx, wid)

    def start_gather(t):
        pltpu.async_copy(table_hbm.at[idx2d.at[t]], rows_v.at[t % 2], sem_g)

    start_gather(0)

    def body(t, carry):
        b = t % 2
        pltpu.make_async_copy(table_hbm.at[idx2d.at[0]],
                              rows_v.at[b], sem_g).wait()

        @pl.when(t >= 1)
        def _():
            pltpu.make_async_copy(rows_v.at[1 - b],
                                  wy_hbm.at[pl.ds(0, _CHUNK)], sem_s).wait()

        @pl.when(t + 1 < _TRIPS)
        def _():
            start_gather(t + 1)

        @pl.when(wid + t * _NW < _NCHUNK)
        def _():
            base = (wid + t * _NW) * _CHUNK
            pltpu.async_copy(rows_v.at[b], wy_hbm.at[pl.ds(base, _CHUNK)], sem_s)

        return carry

    lax.fori_loop(0, _TRIPS, body, 0)

    @pl.when(wid + (_TRIPS - 1) * _NW < _NCHUNK)
    def _():
        pltpu.make_async_copy(rows_v.at[(_TRIPS - 1) % 2],
                              wy_hbm.at[pl.ds(0, _CHUNK)], sem_s).wait()


def _sc_gather(table, senders):
    mesh = plsc.VectorSubcoreMesh(core_axis_name="c", subcore_axis_name="s")
    f = functools.partial(
        pl.kernel,
        mesh=mesh,
        out_type=jax.ShapeDtypeStruct((E, F4), jnp.float32),
        scratch_types=[
            pltpu.VMEM((_TRIPS, _CHUNK), jnp.int32),
            pltpu.VMEM((2, _CHUNK, F4), jnp.float32),
            pltpu.SemaphoreType.DMA,
            pltpu.SemaphoreType.DMA,
            pltpu.SemaphoreType.DMA,
        ],
    )(_sc_gather_body)
    return f(table, senders)


_BN = 2048


def _tc2_body(a_ref, b_ref, o_ref):
    o_ref[...] = a_ref[...] + b_ref[...]


def _tc2(tables):
    nb = _NPAD // _BN
    return pl.pallas_call(
        _tc2_body,
        grid=(nb,),
        in_specs=[
            pl.BlockSpec((_BN, F4), lambda i: (i, 0)),
            pl.BlockSpec((_BN, F4), lambda i, nb=nb: (i + nb, 0)),
        ],
        out_specs=pl.BlockSpec((_BN, F4), lambda i: (i, 0)),
        out_shape=jax.ShapeDtypeStruct((_NPAD, F4), jnp.float32),
    )(tables, tables)


_B3 = 3200


def _tc3_body(x_ref, v_ref, wy_ref, env_ref, w2ax_ref, w2ap_ref, b2a_ref,
              w2b_ref, b2b_ref, w2c_ref, b2c_ref, b4_ref, g1_ref, g2_ref,
              xo_ref, vo_ref):
    xv = x_ref[...]
    Vv = v_ref[...]
    wy = wy_ref[...]
    P = wy * Vv
    h = (jnp.dot(xv, w2ax_ref[...], preferred_element_type=jnp.float32)
         + jnp.dot(P, w2ap_ref[...], preferred_element_type=jnp.float32)
         + b2a_ref[...])
    h = h * jax.nn.sigmoid(h)
    h = jnp.dot(h, w2b_ref[...], preferred_element_type=jnp.float32) + b2b_ref[...]
    h = h * jax.nn.sigmoid(h)
    h = jnp.dot(h, w2c_ref[...], preferred_element_type=jnp.float32) + b2c_ref[...]
    xo_ref[...] = env_ref[...] * h
    t1 = jnp.dot(wy, b4_ref[...], preferred_element_type=jnp.float32) * Vv
    t2 = wy * jnp.dot(Vv, b4_ref[...], preferred_element_type=jnp.float32)
    vo_ref[...] = (jnp.dot(t1, g1_ref[...], preferred_element_type=jnp.float32)
                   + jnp.dot(t2, g2_ref[...], preferred_element_type=jnp.float32))


def _tc3(x, V, wy, env, w2ax, w2ap, b2a, w2b, b2b, w2c, b2c, g1, g2):
    grid = (E // _B3,)
    zero = lambda i: (0, 0)
    return pl.pallas_call(
        _tc3_body,
        grid=grid,
        in_specs=[
            pl.BlockSpec((_B3, D_FEAT), lambda i: (i, 0)),
            pl.BlockSpec((_B3, F4), lambda i: (i, 0)),
            pl.BlockSpec((_B3, F4), lambda i: (i, 0)),
            pl.BlockSpec((_B3, 1), lambda i: (i, 0)),
            pl.BlockSpec((D_FEAT, H), zero),
            pl.BlockSpec((F4, H), zero),
            pl.BlockSpec((1, H), zero),
            pl.BlockSpec((H, H), zero),
            pl.BlockSpec((1, H), zero),
            pl.BlockSpec((H, H), zero),
            pl.BlockSpec((1, H), zero),
            pl.BlockSpec((F4, F4), zero),
            pl.BlockSpec((F4, 3 * MUL), zero),
            pl.BlockSpec((F4, 3 * MUL), zero),
        ],
        out_specs=[
            pl.BlockSpec((_B3, H), lambda i: (i, 0)),
            pl.BlockSpec((_B3, 3 * MUL), lambda i: (i, 0)),
        ],
        out_shape=[
            jax.ShapeDtypeStruct((E, H), jnp.float32),
            jax.ShapeDtypeStruct((E, 3 * MUL), jnp.float32),
        ],
    )(x, V, wy, env, w2ax, w2ap, b2a, w2b, b2b, w2c, b2c,
      jnp.asarray(_B4), g1, g2)


def kernel(vectors, x, V, senders, W1, W2a, b2a, W2b, b2b, W2c, b2c, Wlin):
    w1r = W1 @ jnp.asarray(_Rm)
    w2ax = W2a[:D_FEAT]
    w2ap = EPSILON * (jnp.asarray(_Smat) @ W2a[D_FEAT:])
    scale = EPSILON / np.sqrt(2.0 * MUL)
    wl1 = (Wlin[:MUL] * scale)
    wl2 = (Wlin[MUL:] * scale)
    g1 = jnp.zeros((F4, 3 * MUL), jnp.float32)
    g2 = jnp.zeros((F4, 3 * MUL), jnp.float32)
    rows = _G_ROWS.reshape(3, MUL)
    cols = _G_COLS.reshape(3, MUL)
    for ci in range(3):
        g1 = g1.at[rows[ci][:, None], cols[ci][None, :]].set(wl1)
        g2 = g2.at[rows[ci][:, None], cols[ci][None, :]].set(wl2)
    b2a2 = b2a.reshape(1, H)
    b2b2 = b2b.reshape(1, H)
    b2c2 = b2c.reshape(1, H)
    zeros_stripe = jnp.zeros((_CHUNK, F4), jnp.float32)

    msg, env = _tc1(vectors, x, w1r)
    tables = _sc_scatter(msg, senders, zeros_stripe)
    agg = _tc2(tables)
    wy = _sc_gather(agg, senders)
    x_out, v_out = _tc3(x, V, wy, env, w2ax, w2ap, b2a2, W2b, b2b2,
                        W2c, b2c2, g1, g2)
    return (x_out, v_out)


# speedup vs baseline: 25.0812x; 1.2226x over previous
import functools

import numpy as np
import jax
import jax.numpy as jnp
from jax import lax
from jax.experimental import pallas as pl
from jax.experimental.pallas import tpu as pltpu
from jax.experimental.pallas import tpu_sc as plsc

N_NODES = 10000
E = 160000
D_FEAT = 128
MUL = 32
H = 256
EPSILON = 0.25
CUTOFF = 2.0
F4 = 4 * MUL

_Rm = np.zeros((MUL, F4), np.float32)
_U3s = np.zeros((3, F4), np.float32)
_mask1 = np.zeros((1, F4), np.float32)
_Smat = np.zeros((F4, 2 * MUL), np.float32)
_B4 = np.zeros((F4, F4), np.float32)
for _i in range(MUL):
    _mask1[0, 4 * _i] = 1.0
    _Smat[4 * _i, _i] = 1.0
    for _k in range(4):
        _Rm[_i, 4 * _i + _k] = 1.0
        _B4[4 * _i, 4 * _i + _k] = 1.0
    for _c in range(3):
        _U3s[_c, 4 * _i + 1 + _c] = np.sqrt(3.0)
        _Smat[4 * _i + 1 + _c, MUL + _i] = 1.0 / np.sqrt(3.0)

_G_ROWS = np.array([4 * i + 1 + c for c in range(3) for i in range(MUL)])
_G_COLS = np.array([3 * o + c for c in range(3) for o in range(MUL)])

_P = 6
_C1 = (_P + 1.0) * (_P + 2.0) / 2.0
_C2 = _P * (_P + 2.0)
_C3 = _P * (_P + 1.0) / 2.0

_B1 = 3200


def _tc1_body(vt_ref, x_ref, w1r_ref, u3s_ref, m1_ref, msg_ref, env_ref):
    vt = vt_ref[...]
    yraw = lax.dot_general(vt, u3s_ref[...], (((0,), (0,)), ((), ())),
                           preferred_element_type=jnp.float32)
    l2 = jnp.sum(yraw * yraw, axis=1, keepdims=True) * (1.0 / 96.0)
    ln = jnp.sqrt(l2)
    rinv = 1.0 / (ln + 1e-9)
    yt = yraw * rinv + m1_ref[...]
    w4 = jnp.dot(x_ref[...], w1r_ref[...], preferred_element_type=jnp.float32)
    msg_ref[...] = w4 * yt
    dn = ln * (1.0 / CUTOFF)
    d2 = dn * dn
    d6 = d2 * d2 * d2
    env = jnp.where(dn < 1.0, 1.0 - _C1 * d6 + _C2 * d6 * dn - _C3 * d6 * d2, 0.0)
    env_ref[...] = env


def _tc1(vectors_t, x, w1r):
    grid = (E // _B1,)
    return pl.pallas_call(
        _tc1_body,
        grid=grid,
        in_specs=[
            pl.BlockSpec((3, _B1), lambda i: (0, i)),
            pl.BlockSpec((_B1, D_FEAT), lambda i: (i, 0)),
            pl.BlockSpec((D_FEAT, F4), lambda i: (0, 0)),
            pl.BlockSpec((3, F4), lambda i: (0, 0)),
            pl.BlockSpec((1, F4), lambda i: (0, 0)),
        ],
        out_specs=[
            pl.BlockSpec((_B1, F4), lambda i: (i, 0)),
            pl.BlockSpec((_B1, 1), lambda i: (i, 0)),
        ],
        out_shape=[
            jax.ShapeDtypeStruct((E, F4), jnp.float32),
            jax.ShapeDtypeStruct((E, 1), jnp.float32),
        ],
    )(vectors_t, x, w1r, jnp.asarray(_U3s), jnp.asarray(_mask1))


_CHUNK = 128
_NCHUNK = E // _CHUNK
_NC = 2
_NS = 16
_NW = _NC * _NS
_TRIPS = -(-_NCHUNK // _NW)
_NPAD = 10240
_STRIPE = _NPAD // _NS


def _load_all_idx(send_hbm, idx2d, sem_idx, wid):
    def ibody(t, carry):
        jc = jnp.minimum(wid + t * _NW, _NCHUNK - 1)
        pltpu.async_copy(send_hbm.at[pl.ds(jc * _CHUNK, _CHUNK)],
                         idx2d.at[t], sem_idx)
        return carry

    lax.fori_loop(0, _TRIPS, ibody, 0)

    def iwait(t, carry):
        pltpu.make_async_copy(send_hbm.at[pl.ds(0, _CHUNK)],
                              idx2d.at[t], sem_idx).wait()
        return carry

    lax.fori_loop(0, _TRIPS, iwait, 0)


def _sc_scatter_body(msg_hbm, send_hbm, zeros_hbm, table_hbm,
                     idx2d, rows_v, table_sh, sem_idx, sem_in):
    c = lax.axis_index("c")
    s = lax.axis_index("s")
    wid = s * _NC + c

    pltpu.sync_copy(zeros_hbm, rows_v.at[0])

    def zbody(q, carry):
        pltpu.sync_copy(rows_v.at[0],
                        table_sh.at[pl.ds(s * _STRIPE + q * _CHUNK, _CHUNK)])
        return carry

    lax.fori_loop(0, _STRIPE // _CHUNK, zbody, 0)
    _load_all_idx(send_hbm, idx2d, sem_idx, wid)
    plsc.subcore_barrier()

    def start_in(t):
        jc = jnp.minimum(wid + t * _NW, _NCHUNK - 1)
        pltpu.async_copy(msg_hbm.at[pl.ds(jc * _CHUNK, _CHUNK)],
                         rows_v.at[t % 2], sem_in)

    start_in(0)

    def body(t, carry):
        b = t % 2
        pltpu.make_async_copy(msg_hbm.at[pl.ds(0, _CHUNK)],
                              rows_v.at[b], sem_in).wait()

        @pl.when(t + 1 < _TRIPS)
        def _():
            start_in(t + 1)

        @pl.when(wid + t * _NW < _NCHUNK)
        def _():
            pltpu.sync_copy(rows_v.at[b], table_sh.at[idx2d.at[t]], add=True)

        return carry

    lax.fori_loop(0, _TRIPS, body, 0)
    plsc.subcore_barrier()

    def dbody(q, carry):
        pltpu.sync_copy(table_sh.at[pl.ds(s * _STRIPE + q * _CHUNK, _CHUNK)],
                        rows_v.at[0])
        pltpu.sync_copy(rows_v.at[0],
                        table_hbm.at[pl.ds(c * _NPAD + s * _STRIPE + q * _CHUNK,
                                           _CHUNK)])
        return carry

    lax.fori_loop(0, _STRIPE // _CHUNK, dbody, 0)


def _sc_scatter(msg, senders, zeros_stripe):
    mesh = plsc.VectorSubcoreMesh(core_axis_name="c", subcore_axis_name="s")
    f = functools.partial(
        pl.kernel,
        mesh=mesh,
        out_type=jax.ShapeDtypeStruct((_NC * _NPAD, F4), jnp.float32),
        scratch_types=[
            pltpu.VMEM((_TRIPS, _CHUNK), jnp.int32),
            pltpu.VMEM((2, _CHUNK, F4), jnp.float32),
            pltpu.VMEM_SHARED((_NPAD, F4), jnp.float32),
            pltpu.SemaphoreType.DMA,
            pltpu.SemaphoreType.DMA,
        ],
    )(_sc_scatter_body)
    return f(msg, senders, zeros_stripe)


def _sc_gather_body(table_hbm, send_hbm, wy_hbm, idx2d, rows_v,
                    sem_idx, sem_g, sem_s):
    c = lax.axis_index("c")
    s = lax.axis_index("s")
    wid = s * _NC + c
    _load_all_idx(send_hbm, idx2d, sem_idx, wid)

    def start_gather(t):
        pltpu.async_copy(table_hbm.at[idx2d.at[t]], rows_v.at[t % 2], sem_g)

    start_gather(0)

    def body(t, carry):
        b = t % 2
        pltpu.make_async_copy(table_hbm.at[idx2d.at[0]],
                              rows_v.at[b], sem_g).wait()

        @pl.when(t >= 1)
        def _():
            pltpu.make_async_copy(rows_v.at[1 - b],
                                  wy_hbm.at[pl.ds(0, _CHUNK)], sem_s).wait()

        @pl.when(t + 1 < _TRIPS)
        def _():
            start_gather(t + 1)

        @pl.when(wid + t * _NW < _NCHUNK)
        def _():
            base = (wid + t * _NW) * _CHUNK
            pltpu.async_copy(rows_v.at[b], wy_hbm.at[pl.ds(base, _CHUNK)], sem_s)

        return carry

    lax.fori_loop(0, _TRIPS, body, 0)

    @pl.when(wid + (_TRIPS - 1) * _NW < _NCHUNK)
    def _():
        pltpu.make_async_copy(rows_v.at[(_TRIPS - 1) % 2],
                              wy_hbm.at[pl.ds(0, _CHUNK)], sem_s).wait()


def _sc_gather(table, senders):
    mesh = plsc.VectorSubcoreMesh(core_axis_name="c", subcore_axis_name="s")
    f = functools.partial(
        pl.kernel,
        mesh=mesh,
        out_type=jax.ShapeDtypeStruct((E, F4), jnp.float32),
        scratch_types=[
            pltpu.VMEM((_TRIPS, _CHUNK), jnp.int32),
            pltpu.VMEM((2, _CHUNK, F4), jnp.float32),
            pltpu.SemaphoreType.DMA,
            pltpu.SemaphoreType.DMA,
            pltpu.SemaphoreType.DMA,
        ],
    )(_sc_gather_body)
    return f(table, senders)


_BN = 2048


def _tc2_body(a_ref, b_ref, o_ref):
    o_ref[...] = a_ref[...] + b_ref[...]


def _tc2(tables):
    nb = _NPAD // _BN
    return pl.pallas_call(
        _tc2_body,
        grid=(nb,),
        in_specs=[
            pl.BlockSpec((_BN, F4), lambda i: (i, 0)),
            pl.BlockSpec((_BN, F4), lambda i, nb=nb: (i + nb, 0)),
        ],
        out_specs=pl.BlockSpec((_BN, F4), lambda i: (i, 0)),
        out_shape=jax.ShapeDtypeStruct((_NPAD, F4), jnp.float32),
    )(tables, tables)


_B3 = 3200


def _tc3_body(x_ref, v_ref, wy_ref, env_ref, w2ax_ref, w2ap_ref, b2a_ref,
              w2b_ref, b2b_ref, w2c_ref, b2c_ref, b4_ref, g1_ref, g2_ref,
              xo_ref, vo_ref):
    xv = x_ref[...]
    Vv = v_ref[...]
    wy = wy_ref[...]
    P = wy * Vv
    h = (jnp.dot(xv, w2ax_ref[...], preferred_element_type=jnp.float32)
         + jnp.dot(P, w2ap_ref[...], preferred_element_type=jnp.float32)
         + b2a_ref[...])
    h = h * jax.nn.sigmoid(h)
    h = jnp.dot(h, w2b_ref[...], preferred_element_type=jnp.float32) + b2b_ref[...]
    h = h * jax.nn.sigmoid(h)
    h = jnp.dot(h, w2c_ref[...], preferred_element_type=jnp.float32) + b2c_ref[...]
    xo_ref[...] = env_ref[...] * h
    t1 = jnp.dot(wy, b4_ref[...], preferred_element_type=jnp.float32) * Vv
    t2 = wy * jnp.dot(Vv, b4_ref[...], preferred_element_type=jnp.float32)
    vo = (jnp.dot(t1, g1_ref[...], preferred_element_type=jnp.float32)
          + jnp.dot(t2, g2_ref[...], preferred_element_type=jnp.float32))
    vo_ref[...] = vo.T


def _tc3(x, V, wy, env, w2ax, w2ap, b2a, w2b, b2b, w2c, b2c, g1, g2):
    grid = (E // _B3,)
    zero = lambda i: (0, 0)
    return pl.pallas_call(
        _tc3_body,
        grid=grid,
        in_specs=[
            pl.BlockSpec((_B3, D_FEAT), lambda i: (i, 0)),
            pl.BlockSpec((_B3, F4), lambda i: (i, 0)),
            pl.BlockSpec((_B3, F4), lambda i: (i, 0)),
            pl.BlockSpec((_B3, 1), lambda i: (i, 0)),
            pl.BlockSpec((D_FEAT, H), zero),
            pl.BlockSpec((F4, H), zero),
            pl.BlockSpec((1, H), zero),
            pl.BlockSpec((H, H), zero),
            pl.BlockSpec((1, H), zero),
            pl.BlockSpec((H, H), zero),
            pl.BlockSpec((1, H), zero),
            pl.BlockSpec((F4, F4), zero),
            pl.BlockSpec((F4, 3 * MUL), zero),
            pl.BlockSpec((F4, 3 * MUL), zero),
        ],
        out_specs=[
            pl.BlockSpec((_B3, H), lambda i: (i, 0)),
            pl.BlockSpec((3 * MUL, _B3), lambda i: (0, i)),
        ],
        out_shape=[
            jax.ShapeDtypeStruct((E, H), jnp.float32),
            jax.ShapeDtypeStruct((3 * MUL, E), jnp.float32),
        ],
    )(x, V, wy, env, w2ax, w2ap, b2a, w2b, b2b, w2c, b2c,
      jnp.asarray(_B4), g1, g2)


def kernel(vectors, x, V, senders, W1, W2a, b2a, W2b, b2b, W2c, b2c, Wlin):
    w1r = W1 @ jnp.asarray(_Rm)
    w2ax = W2a[:D_FEAT]
    w2ap = EPSILON * (jnp.asarray(_Smat) @ W2a[D_FEAT:])
    scale = EPSILON / np.sqrt(2.0 * MUL)
    wl1 = (Wlin[:MUL] * scale)
    wl2 = (Wlin[MUL:] * scale)
    g1 = jnp.zeros((F4, 3 * MUL), jnp.float32)
    g2 = jnp.zeros((F4, 3 * MUL), jnp.float32)
    rows = _G_ROWS.reshape(3, MUL)
    cols = _G_COLS.reshape(3, MUL)
    for ci in range(3):
        g1 = g1.at[rows[ci][:, None], cols[ci][None, :]].set(wl1)
        g2 = g2.at[rows[ci][:, None], cols[ci][None, :]].set(wl2)
    b2a2 = b2a.reshape(1, H)
    b2b2 = b2b.reshape(1, H)
    b2c2 = b2c.reshape(1, H)
    zeros_stripe = jnp.zeros((_CHUNK, F4), jnp.float32)

    msg, env = _tc1(vectors.T, x, w1r)
    tables = _sc_scatter(msg, senders, zeros_stripe)
    agg = _tc2(tables)
    wy = _sc_gather(agg, senders)
    x_out, v_out_t = _tc3(x, V, wy, env, w2ax, w2ap, b2a2, W2b, b2b2,
                          W2c, b2c2, g1, g2)
    return (x_out, v_out_t.T)
